# baseline (speedup 1.0000x reference)
import jax
import jax.numpy as jnp
from jax import lax
from jax.experimental import pallas as pl
from jax.experimental.pallas import tpu as pltpu
from jax.experimental.pallas import tpu_sc as plsc

B, S, H, V, P, T = 32, 512, 768, 30522, 512, 2
EPS = 1e-12
L = 16
NW = 32
SS = S // NW
RB = 512
MAGIC = 0x5F3759DF


def _sc_body(ids2, am_flat, word_hbm, out_hbm, mask_hbm,
             rowidx_v, rawids_v, idsall_v, amall_v, maskall_v,
             b0, b1, b2, b3, g0, g1, g2, g3, w0, w1, w2, w3, sst):
    wid = lax.axis_index("s") * 2 + lax.axis_index("c")
    base_s = wid * SS
    bufs = (b0, b1, b2, b3)
    gsem = (g0, g1, g2, g3)
    wsem = (w0, w1, w2, w3)
    lane = lax.iota(jnp.int32, L)

    rowhi = wid // 8
    col = (wid % 8) * L
    rowidx_v[pl.ds(0, L)] = lane * 4 + rowhi
    rowidx_v[pl.ds(L, L)] = (lane + L) * 4 + rowhi
    pltpu.async_copy(ids2.at[rowidx_v], rawids_v, sst).wait()
    pltpu.sync_copy(am_flat.at[pl.ds(wid * S, S)], amall_v)

    def extract(k, c):
        idsall_v[k // 2, pl.ds((k % 2) * L, L)] = rawids_v[k, pl.ds(col, L)]
        return c
    lax.fori_loop(0, B, extract, 0)

    def mask_row(k, c):
        maskall_v[pl.ds(k * L, L)] = (1.0 - amall_v[pl.ds(k * L, L)]) * -10000.0
        return c
    lax.fori_loop(0, S // L, mask_row, 0)

    pltpu.async_copy(word_hbm.at[idsall_v.at[0]], b0, g0)
    pltpu.async_copy(word_hbm.at[idsall_v.at[1]], b1, g1)

    NC = B // 2

    def k_body(k, c0):
        for r in range(4):
            c = k * 4 + r
            rr = (r + 2) % 4

            @pl.when(c >= 2)
            def _():
                pltpu.make_async_copy(word_hbm.at[pl.ds(0, 2 * SS)],
                                      bufs[rr], wsem[rr]).wait()

            @pl.when(c + 2 < NC)
            def _():
                pltpu.async_copy(word_hbm.at[idsall_v.at[c + 2]], bufs[rr],
                                 gsem[rr])

            pltpu.make_async_copy(word_hbm.at[pl.ds(0, 2 * SS)], bufs[r],
                                  gsem[r]).wait()
            tok0 = (2 * c) * S + base_s
            tok1 = (2 * c + 1) * S + base_s
            pltpu.async_copy(bufs[r].at[pl.ds(0, SS)],
                             out_hbm.at[pl.ds(tok0, SS)], wsem[r])
            pltpu.async_copy(bufs[r].at[pl.ds(SS, SS)],
                             out_hbm.at[pl.ds(tok1, SS)], wsem[r])
        return c0
    lax.fori_loop(0, NC // 4, k_body, 0)

    pltpu.make_async_copy(word_hbm.at[pl.ds(0, 2 * SS)], b2, w2).wait()
    pltpu.make_async_copy(word_hbm.at[pl.ds(0, 2 * SS)], b3, w3).wait()
    pltpu.sync_copy(maskall_v, mask_hbm.at[pl.ds(wid * S, S)])


@jax.jit
def _sc_gather(ids2, am_flat, word):
    mesh = plsc.VectorSubcoreMesh(core_axis_name="c", subcore_axis_name="s")
    k = pl.kernel(
        _sc_body, mesh=mesh,
        out_type=(jax.ShapeDtypeStruct((B * S, H), jnp.float32),
                  jax.ShapeDtypeStruct((B * S,), jnp.float32)),
        scratch_types=[
            pltpu.VMEM((NW,), jnp.int32),
            pltpu.VMEM((NW, 128), jnp.int32),
            pltpu.VMEM((NW // 2, 2 * L), jnp.int32),
            pltpu.VMEM((S,), jnp.float32),
            pltpu.VMEM((S,), jnp.float32),
            pltpu.VMEM((2 * SS, H), jnp.float32),
            pltpu.VMEM((2 * SS, H), jnp.float32),
            pltpu.VMEM((2 * SS, H), jnp.float32),
            pltpu.VMEM((2 * SS, H), jnp.float32),
            pltpu.SemaphoreType.DMA,
            pltpu.SemaphoreType.DMA,
            pltpu.SemaphoreType.DMA,
            pltpu.SemaphoreType.DMA,
            pltpu.SemaphoreType.DMA,
            pltpu.SemaphoreType.DMA,
            pltpu.SemaphoreType.DMA,
            pltpu.SemaphoreType.DMA,
            pltpu.SemaphoreType.DMA,
        ],
    )
    return k(ids2, am_flat, word)


def _ln_body(we_ref, pe_ref, ttf_ref, te2_ref, gam_ref, bet_ref, out_ref):
    e = (we_ref[...] + pe_ref[...] + te2_ref[0:1, :]
         + ttf_ref[...] * (te2_ref[1:2, :] - te2_ref[0:1, :]))
    mean = jnp.mean(e, axis=1, keepdims=True)
    var = jnp.mean(jnp.square(e - mean), axis=1, keepdims=True)
    out_ref[...] = ((e - mean) * lax.rsqrt(var + EPS) * gam_ref[...]
                    + bet_ref[...])


@jax.jit
def _tc_ln(we_flat, pe, ttf, te2, gamma, beta):
    grid = (S // RB, B)
    return pl.pallas_call(
        _ln_body,
        grid=grid,
        in_specs=[
            pl.BlockSpec((RB, H), lambda pc, b: (b * (S // RB) + pc, 0)),
            pl.BlockSpec((RB, H), lambda pc, b: (pc, 0)),
            pl.BlockSpec((RB, 1), lambda pc, b: (b * (S // RB) + pc, 0)),
            pl.BlockSpec((T, H), lambda pc, b: (0, 0)),
            pl.BlockSpec((1, H), lambda pc, b: (0, 0)),
            pl.BlockSpec((1, H), lambda pc, b: (0, 0)),
        ],
        out_specs=pl.BlockSpec((RB, H), lambda pc, b: (b * (S // RB) + pc, 0)),
        out_shape=jax.ShapeDtypeStruct((B * S, H), jnp.float32),
    )(we_flat, pe, ttf, te2, gamma, beta)


def kernel(input_ids, attention_mask, token_type_ids, word_embeddings,
           position_embeddings, token_type_embeddings, ln_gamma, ln_beta):
    ids2 = input_ids.astype(jnp.int32).reshape(B * S // 128, 128)
    am_flat = attention_mask.astype(jnp.float32).reshape(B * S)
    we_flat, mask_flat = _sc_gather(ids2, am_flat, word_embeddings)
    ttf = token_type_ids.astype(jnp.float32).reshape(B * S, 1)
    out_flat = _tc_ln(we_flat, position_embeddings, ttf,
                      token_type_embeddings, ln_gamma.reshape(1, H),
                      ln_beta.reshape(1, H))
    return out_flat.reshape(B, S, H), mask_flat.reshape(B, S)
